# Initial kernel scaffold; baseline (speedup 1.0000x reference)
import jax
import jax.numpy as jnp
from jax.experimental import pallas as pl


def kernel(x, edge_index, W_self1, W_neigh1, b1, W_self2, W_neigh2, b2, W_self3, W_neigh3, b3):
    raise NotImplementedError("write your pallas kernel here")


# speedup vs baseline: 4.4582x; 4.4582x over previous
import functools

import jax
import jax.numpy as jnp
from jax import lax
from jax.experimental import pallas as pl
from jax.experimental.pallas import tpu as pltpu
from jax.experimental.pallas import tpu_sc as plsc

N_NODES = 10000
N_EDGES = 320000
N_PAD = 10240
ROWS_PER_TILE = 640
CHUNK = 128
N_CHUNKS = N_EDGES // CHUNK
N_SUBCORES = 16


def _sc_agg_l1_body(table, src, dst, zblk,
                    agg_out,
                    src_v, dst_v, rows_v, agg_sh, sem):
    c = lax.axis_index("c")
    s = lax.axis_index("s")

    pltpu.sync_copy(zblk, agg_sh.at[pl.ds(s * ROWS_PER_TILE, ROWS_PER_TILE)])
    plsc.subcore_barrier()

    half = N_CHUNKS // 2
    n_my = (half - s + N_SUBCORES - 1) // N_SUBCORES

    def it(i, carry):
        base = (c * half + s + i * N_SUBCORES) * CHUNK
        pltpu.sync_copy(src.at[pl.ds(base, CHUNK)], src_v)
        pltpu.sync_copy(dst.at[pl.ds(base, CHUNK)], dst_v.at[0])
        pltpu.async_copy(table.at[src_v], rows_v, sem).wait()
        pltpu.sync_copy(rows_v, agg_sh.at[dst_v.at[0]], add=True)
        return carry

    lax.fori_loop(0, n_my, it, 0)
    plsc.subcore_barrier()

    row0 = c * N_PAD + s * ROWS_PER_TILE
    pltpu.sync_copy(agg_sh.at[pl.ds(s * ROWS_PER_TILE, ROWS_PER_TILE)],
                    agg_out.at[pl.ds(row0, ROWS_PER_TILE)])


_SC_AGG_L1 = pl.kernel(
    _sc_agg_l1_body,
    out_type=jax.ShapeDtypeStruct((2 * N_PAD, 128), jnp.float32),
    mesh=plsc.VectorSubcoreMesh(core_axis_name="c", subcore_axis_name="s"),
    scratch_types=(
        pltpu.VMEM((CHUNK,), jnp.int32),
        pltpu.VMEM((1, CHUNK), jnp.int32),
        pltpu.VMEM((CHUNK, 128), jnp.float32),
        pltpu.VMEM_SHARED((N_PAD, 128), jnp.float32),
        pltpu.SemaphoreType.DMA,
    ),
)


def _sc_deg_body(dst, ones_in, zblk,
                 deg_out,
                 dst_v, ones_v, deg_sh, sem):
    c = lax.axis_index("c")
    s = lax.axis_index("s")

    pltpu.sync_copy(zblk, deg_sh.at[pl.ds(s * ROWS_PER_TILE, ROWS_PER_TILE)])
    pltpu.sync_copy(ones_in, ones_v)
    plsc.subcore_barrier()

    half = N_CHUNKS // 2
    n_my = (half - s + N_SUBCORES - 1) // N_SUBCORES

    def it(i, carry):
        base = (c * half + s + i * N_SUBCORES) * CHUNK
        pltpu.sync_copy(dst.at[pl.ds(base, CHUNK)], dst_v.at[0])
        pltpu.sync_copy(ones_v, deg_sh.at[dst_v.at[0]], add=True)
        return carry

    lax.fori_loop(0, n_my, it, 0)
    plsc.subcore_barrier()

    row0 = c * N_PAD + s * ROWS_PER_TILE
    pltpu.sync_copy(deg_sh.at[pl.ds(s * ROWS_PER_TILE, ROWS_PER_TILE)],
                    deg_out.at[pl.ds(row0, ROWS_PER_TILE)])


_SC_DEG = pl.kernel(
    _sc_deg_body,
    out_type=jax.ShapeDtypeStruct((2 * N_PAD, 128), jnp.float32),
    mesh=plsc.VectorSubcoreMesh(core_axis_name="c", subcore_axis_name="s"),
    scratch_types=(
        pltpu.VMEM((1, CHUNK), jnp.int32),
        pltpu.VMEM((CHUNK, 128), jnp.float32),
        pltpu.VMEM_SHARED((N_PAD, 128), jnp.float32),
        pltpu.SemaphoreType.DMA,
    ),
)


def _sc_agg_h_body(table, srcx, dst,  zblk,
                   agg_out,
                   src_v, dst_v, rows_v, agg_sh, sem):
    c = lax.axis_index("c")
    s = lax.axis_index("s")

    pltpu.sync_copy(zblk, agg_sh.at[pl.ds(s * ROWS_PER_TILE, ROWS_PER_TILE)])
    plsc.subcore_barrier()

    n_my = (N_CHUNKS - s + N_SUBCORES - 1) // N_SUBCORES

    def it(i, carry):
        base = (s + i * N_SUBCORES) * CHUNK
        pltpu.sync_copy(srcx.at[pl.ds(c * N_EDGES + base, CHUNK)], src_v)
        pltpu.sync_copy(dst.at[pl.ds(base, CHUNK)], dst_v.at[0])
        pltpu.async_copy(table.at[src_v], rows_v, sem).wait()
        pltpu.sync_copy(rows_v, agg_sh.at[dst_v.at[0]], add=True)
        return carry

    lax.fori_loop(0, n_my, it, 0)
    plsc.subcore_barrier()

    row0 = c * N_PAD + s * ROWS_PER_TILE
    pltpu.sync_copy(agg_sh.at[pl.ds(s * ROWS_PER_TILE, ROWS_PER_TILE)],
                    agg_out.at[pl.ds(row0, ROWS_PER_TILE)])


_SC_AGG_H = pl.kernel(
    _sc_agg_h_body,
    out_type=jax.ShapeDtypeStruct((2 * N_PAD, 128), jnp.float32),
    mesh=plsc.VectorSubcoreMesh(core_axis_name="c", subcore_axis_name="s"),
    scratch_types=(
        pltpu.VMEM((CHUNK,), jnp.int32),
        pltpu.VMEM((1, CHUNK), jnp.int32),
        pltpu.VMEM((CHUNK, 128), jnp.float32),
        pltpu.VMEM_SHARED((N_PAD, 128), jnp.float32),
        pltpu.SemaphoreType.DMA,
    ),
)


def _tc_l1_body(h, p0, p1, d0, d1, ws, wn, b, out, deg_out):
    degsum = d0[...] + d1[...]
    inv = 1.0 / jnp.maximum(degsum, 1.0)
    dot = functools.partial(jnp.dot, preferred_element_type=jnp.float32,
                            precision=lax.Precision.HIGHEST)
    acc = dot(h[...], ws[...]) + dot((p0[...] + p1[...]) * inv, wn[...])
    acc += b[...]
    acc = jnp.maximum(acc, 0.0)
    out[0] = acc[:, :128]
    out[1] = acc[:, 128:]
    deg_out[...] = degsum


def _tc_layer_body(relu, split_out, h0, h1, a0, a1, deg, ws0, ws1, wn0, wn1, b,
                   out):
    inv = 1.0 / jnp.maximum(deg[...], 1.0)
    dot = functools.partial(jnp.dot, preferred_element_type=jnp.float32,
                            precision=lax.Precision.HIGHEST)
    acc = dot(h0[...], ws0[...]) + dot(h1[...], ws1[...])
    acc += dot(a0[...] * inv, wn0[...]) + dot(a1[...] * inv, wn1[...])
    acc += b[...]
    if relu:
        acc = jnp.maximum(acc, 0.0)
    if split_out:
        out[0] = acc[:, :128]
        out[1] = acc[:, 128:]
    else:
        out[...] = acc


_BM = 1000


def _make_tc_l1():
    bm = _BM
    in_specs = [
        pl.BlockSpec((bm, 128), lambda m: (m, 0)),
        pl.BlockSpec((bm, 128), lambda m: (m, 0)),
        pl.BlockSpec((bm, 128), lambda m: (m, 0)),
        pl.BlockSpec((bm, 1), lambda m: (m, 0)),
        pl.BlockSpec((bm, 1), lambda m: (m, 0)),
        pl.BlockSpec((128, 256), lambda m: (0, 0)),
        pl.BlockSpec((128, 256), lambda m: (0, 0)),
        pl.BlockSpec((1, 256), lambda m: (0, 0)),
    ]
    return pl.pallas_call(
        _tc_l1_body,
        grid=(N_NODES // bm,),
        in_specs=in_specs,
        out_specs=(pl.BlockSpec((2, bm, 128), lambda m: (0, m, 0)),
                   pl.BlockSpec((bm, 1), lambda m: (m, 0))),
        out_shape=(jax.ShapeDtypeStruct((2, N_NODES, 128), jnp.float32),
                   jax.ShapeDtypeStruct((N_NODES, 1), jnp.float32)),
    )


def _make_tc_layer(relu, split_out):
    bm = _BM
    in_specs = [
        pl.BlockSpec((bm, 128), lambda m: (m, 0)),
        pl.BlockSpec((bm, 128), lambda m: (m, 0)),
        pl.BlockSpec((bm, 128), lambda m: (m, 0)),
        pl.BlockSpec((bm, 128), lambda m: (m, 0)),
        pl.BlockSpec((bm, 1), lambda m: (m, 0)),
        pl.BlockSpec((128, 256), lambda m: (0, 0)),
        pl.BlockSpec((128, 256), lambda m: (0, 0)),
        pl.BlockSpec((128, 256), lambda m: (0, 0)),
        pl.BlockSpec((128, 256), lambda m: (0, 0)),
        pl.BlockSpec((1, 256), lambda m: (0, 0)),
    ]
    if split_out:
        out_shape = jax.ShapeDtypeStruct((2, N_NODES, 128), jnp.float32)
        out_spec = pl.BlockSpec((2, bm, 128), lambda m: (0, m, 0))
    else:
        out_shape = jax.ShapeDtypeStruct((N_NODES, 256), jnp.float32)
        out_spec = pl.BlockSpec((bm, 256), lambda m: (m, 0))
    return pl.pallas_call(
        functools.partial(_tc_layer_body, relu, split_out),
        grid=(N_NODES // bm,),
        in_specs=in_specs,
        out_specs=out_spec,
        out_shape=out_shape,
    )


_TC_L1 = _make_tc_l1()
_TC_L2 = _make_tc_layer(True, True)
_TC_L3 = _make_tc_layer(False, False)


def kernel(x, edge_index, W_self1, W_neigh1, b1, W_self2, W_neigh2, b2,
           W_self3, W_neigh3, b3):
    n = N_NODES
    src = edge_index[0].astype(jnp.int32)
    dst = edge_index[1].astype(jnp.int32)
    srcx = jnp.concatenate([src, src + n])

    z128 = jnp.zeros((ROWS_PER_TILE, 128), jnp.float32)
    ones128 = jnp.ones((CHUNK, 128), jnp.float32)

    degp = _SC_DEG(dst, ones128, z128)
    agg1 = _SC_AGG_L1(x, src, dst, z128)
    h, degc = _TC_L1(x, agg1[:n], agg1[N_PAD:N_PAD + n],
                     degp[:n, 0:1], degp[N_PAD:N_PAD + n, 0:1],
                     W_self1, W_neigh1, b1.reshape(1, -1))
    h2 = h.reshape(2 * n, 128)

    agg2 = _SC_AGG_H(h2, srcx, dst, z128)
    h = _TC_L2(h2[:n], h2[n:], agg2[:n], agg2[N_PAD:N_PAD + n], degc,
               W_self2[:128], W_self2[128:], W_neigh2[:128], W_neigh2[128:],
               b2.reshape(1, -1))
    h3 = h.reshape(2 * n, 128)

    agg3 = _SC_AGG_H(h3, srcx, dst, z128)
    out = _TC_L3(h3[:n], h3[n:], agg3[:n], agg3[N_PAD:N_PAD + n], degc,
                 W_self3[:128], W_self3[128:], W_neigh3[:128], W_neigh3[128:],
                 b3.reshape(1, -1))
    return out
